# baseline (speedup 1.0000x reference)
import jax
import jax.numpy as jnp
from jax import lax
from jax.experimental import pallas as pl
from jax.experimental.pallas import tpu as pltpu
from jax.experimental.pallas import tpu_sc as plsc

B = 1024
FAN0 = 25
FAN1 = 10
D = 128
HID = 128

NC = 2
NS = 16
NW = NC * NS

M_ROWS = B + B * FAN1
MR_PER_W = M_ROWS // NW
M_CHUNK = 8
M_CHUNKS = MR_PER_W // M_CHUNK
N1_CHUNKS = B // M_CHUNK

H_ROWS = B * FAN1
HR_PER_W = H_ROWS // NW
H_GLEN = 64
H_GATHERS = HR_PER_W // H_GLEN


def _sc_gather_body(feats_hbm, n1_hbm, n2n_hbm, idxh_hbm,
                    m1_out, m2_out, hp2_out,
                    idxv0, idxv1, buf0, buf1, obuf, idxh, bufh,
                    msem0, msem1, hsem):
    wid = lax.axis_index("s") * NC + lax.axis_index("c")
    idxv = (idxv0, idxv1)
    buf = (buf0, buf1)
    msem = (msem0, msem1)

    m_row0 = wid * MR_PER_W
    h_row0 = wid * HR_PER_W

    pltpu.sync_copy(idxh_hbm.at[pl.ds(h_row0, HR_PER_W)], idxh)
    for j in range(H_GATHERS):
        pltpu.async_copy(feats_hbm.at[idxh.at[pl.ds(j * H_GLEN, H_GLEN)]],
                         bufh.at[pl.ds(j * H_GLEN, H_GLEN)], hsem)

    def fire_m(c, b):
        g = wid * M_CHUNKS + c

        @pl.when(g < N1_CHUNKS)
        def _():
            pltpu.sync_copy(n1_hbm.at[pl.ds(g * M_CHUNK, M_CHUNK)], idxv[b])

        @pl.when(g >= N1_CHUNKS)
        def _():
            pltpu.sync_copy(n2n_hbm.at[pl.ds(g * M_CHUNK - B, M_CHUNK)],
                            idxv[b])

        for r in range(M_CHUNK):
            pltpu.async_copy(feats_hbm.at[idxv[b].at[r]], buf[b].at[r],
                             msem[b])

    def drain_m(b):
        for r in range(M_CHUNK):
            pltpu.make_async_copy(feats_hbm.at[idxv[b].at[r]], buf[b].at[r],
                                  msem[b]).wait()

    fire_m(0, 0)
    fire_m(1, 1)

    def reduce_chunk(b):
        for r in range(M_CHUNK):
            accs = [buf[b][r, 0, pl.ds(16 * j, 16)] for j in range(D // 16)]

            def acc_body(k, a):
                return tuple(a[j] + buf[b][r, k, pl.ds(16 * j, 16)]
                             for j in range(D // 16))

            accs = lax.fori_loop(1, FAN0, acc_body, tuple(accs))
            for j in range(D // 16):
                obuf[r, pl.ds(16 * j, 16)] = accs[j] * (1.0 / FAN0)

    def write_chunk(c):
        row = m_row0 + c * M_CHUNK

        @pl.when(row < B)
        def _():
            pltpu.sync_copy(obuf, m1_out.at[pl.ds(row, M_CHUNK)])

        @pl.when(row >= B)
        def _():
            pltpu.sync_copy(obuf, m2_out.at[pl.ds(row - B, M_CHUNK)])

    def pair_body(g, carry):
        for b in range(2):
            c = 2 * g + b
            drain_m(b)
            reduce_chunk(b)
            write_chunk(c)

            @pl.when(c + 2 < M_CHUNKS)
            def _():
                fire_m(c + 2, b)
        return carry

    lax.fori_loop(0, M_CHUNKS // 2, pair_body, 0)

    for j in range(H_GATHERS):
        pltpu.make_async_copy(feats_hbm.at[idxh.at[pl.ds(j * H_GLEN, H_GLEN)]],
                              bufh.at[pl.ds(j * H_GLEN, H_GLEN)], hsem).wait()
    pltpu.sync_copy(bufh, hp2_out.at[pl.ds(h_row0, HR_PER_W)])


@jax.jit
def _sc_gather(feats, n1, n2n, idx_h):
    mesh = plsc.VectorSubcoreMesh(core_axis_name="c", subcore_axis_name="s",
                                  num_cores=NC, num_subcores=NS)
    return pl.kernel(
        _sc_gather_body,
        out_type=(
            jax.ShapeDtypeStruct((B, D), jnp.float32),
            jax.ShapeDtypeStruct((B * FAN1, D), jnp.float32),
            jax.ShapeDtypeStruct((H_ROWS, D), jnp.float32),
        ),
        mesh=mesh,
        scratch_types=[
            pltpu.VMEM((M_CHUNK, FAN0), jnp.int32),
            pltpu.VMEM((M_CHUNK, FAN0), jnp.int32),
            pltpu.VMEM((M_CHUNK, FAN0, D), jnp.float32),
            pltpu.VMEM((M_CHUNK, FAN0, D), jnp.float32),
            pltpu.VMEM((M_CHUNK, D), jnp.float32),
            pltpu.VMEM((HR_PER_W,), jnp.int32),
            pltpu.VMEM((HR_PER_W, D), jnp.float32),
            pltpu.SemaphoreType.DMA,
            pltpu.SemaphoreType.DMA,
            pltpu.SemaphoreType.DMA,
        ],
    )(feats, n1, n2n, idx_h)


TC_NCH = 4
TC_CH = H_ROWS // TC_NCH
TC_BB = B // TC_NCH


def _tc_dense_body(x_ref, m1_ref, hp2_hbm, m2_hbm, w0_ref, b0_ref,
                   w1_ref, b1_ref, out_ref,
                   hbuf0, hbuf1, mbuf0, mbuf1, na_s, sem0, sem1):
    hbuf = (hbuf0, hbuf1)
    mbuf = (mbuf0, mbuf1)
    sem = (sem0, sem1)

    def lin(a, b, w_ref, bias):
        return jnp.maximum(
            jnp.dot(a, w_ref[:D], preferred_element_type=jnp.float32)
            + jnp.dot(b, w_ref[D:], preferred_element_type=jnp.float32)
            + bias, 0.0)

    def start(i, b):
        pltpu.async_copy(hp2_hbm.at[pl.ds(i * TC_CH, TC_CH)], hbuf[b], sem[b])
        pltpu.async_copy(m2_hbm.at[pl.ds(i * TC_CH, TC_CH)], mbuf[b], sem[b])

    def wait(i, b):
        pltpu.make_async_copy(hp2_hbm.at[pl.ds(i * TC_CH, TC_CH)], hbuf[b],
                              sem[b]).wait()
        pltpu.make_async_copy(m2_hbm.at[pl.ds(i * TC_CH, TC_CH)], mbuf[b],
                              sem[b]).wait()

    start(0, 0)
    start(1, 1)
    h0 = lin(x_ref[...], m1_ref[...], w0_ref, b0_ref[...])
    for i in range(TC_NCH):
        b = i % 2
        wait(i, b)
        h2 = lin(hbuf[b][...], mbuf[b][...], w0_ref, b0_ref[...])
        na_s[pl.ds(i * TC_BB, TC_BB), :] = jnp.mean(
            h2.reshape(TC_BB, FAN1, HID), axis=1)
        if i + 2 < TC_NCH:
            start(i + 2, b)
    out_ref[...] = lin(h0, na_s[...], w1_ref, b1_ref[...])


@jax.jit
def _tc_dense(x, m1, hp2, m2, W0, b0, W1, b1):
    anyspec = pl.BlockSpec(memory_space=pltpu.MemorySpace.HBM)
    vmem = pl.BlockSpec(memory_space=pltpu.MemorySpace.VMEM)
    return pl.pallas_call(
        _tc_dense_body,
        in_specs=[vmem, vmem, anyspec, anyspec, vmem, vmem, vmem, vmem],
        out_specs=vmem,
        out_shape=jax.ShapeDtypeStruct((B, HID), jnp.float32),
        scratch_shapes=[
            pltpu.VMEM((TC_CH, D), jnp.float32),
            pltpu.VMEM((TC_CH, D), jnp.float32),
            pltpu.VMEM((TC_CH, D), jnp.float32),
            pltpu.VMEM((TC_CH, D), jnp.float32),
            pltpu.VMEM((B, HID), jnp.float32),
            pltpu.SemaphoreType.DMA,
            pltpu.SemaphoreType.DMA,
        ],
    )(x, m1, hp2, m2, W0, b0.reshape(1, HID), W1, b1.reshape(1, HID))


def kernel(x, nodes, feats, neighs1, neighs2, neighs2_neighs, W0, b0, W1, b1):
    idx_h = neighs2.reshape(-1).astype(jnp.int32)
    m1, m2, hp2 = _sc_gather(feats, neighs1.astype(jnp.int32),
                             neighs2_neighs.astype(jnp.int32), idx_h)
    out = _tc_dense(x, m1, hp2, m2, W0, b0, W1, b1)
    return out.reshape(B, 1, HID)


# speedup vs baseline: 1.0103x; 1.0103x over previous
import jax
import jax.numpy as jnp
from jax import lax
from jax.experimental import pallas as pl
from jax.experimental.pallas import tpu as pltpu
from jax.experimental.pallas import tpu_sc as plsc

B = 1024
FAN0 = 25
FAN1 = 10
D = 128
HID = 128

NC = 2
NS = 16
NW = NC * NS

M_ROWS = B + B * FAN1
MR_PER_W = M_ROWS // NW
M_CHUNK = 8
M_CHUNKS = MR_PER_W // M_CHUNK
N1_CHUNKS = B // M_CHUNK

H_ROWS = B * FAN1
HR_PER_W = H_ROWS // NW
H_GLEN = 64
H_GATHERS = HR_PER_W // H_GLEN


def _sc_gather_body(feats_hbm, n1_hbm, n2n_hbm, idxh_hbm,
                    m1_out, m2_out, hp2_out,
                    idxv0, idxv1, buf0, buf1, obuf, idxh, bufh,
                    msem0, msem1, hsem):
    wid = lax.axis_index("s") * NC + lax.axis_index("c")
    idxv = (idxv0, idxv1)
    buf = (buf0, buf1)
    msem = (msem0, msem1)

    m_row0 = wid * MR_PER_W
    h_row0 = wid * HR_PER_W

    pltpu.sync_copy(idxh_hbm.at[pl.ds(h_row0, HR_PER_W)], idxh)
    for j in range(H_GATHERS):
        pltpu.async_copy(feats_hbm.at[idxh.at[pl.ds(j * H_GLEN, H_GLEN)]],
                         bufh.at[pl.ds(j * H_GLEN, H_GLEN)], hsem)

    def fire_m(c, b):
        g = wid * M_CHUNKS + c

        @pl.when(g < N1_CHUNKS)
        def _():
            pltpu.sync_copy(n1_hbm.at[pl.ds(g * M_CHUNK, M_CHUNK)], idxv[b])

        @pl.when(g >= N1_CHUNKS)
        def _():
            pltpu.sync_copy(n2n_hbm.at[pl.ds(g * M_CHUNK - B, M_CHUNK)],
                            idxv[b])

        for r in range(M_CHUNK):
            pltpu.async_copy(feats_hbm.at[idxv[b].at[r]],
                             buf[b].at[pl.ds(r * FAN0, FAN0)], msem[b])

    def drain_m(b):
        pltpu.make_async_copy(feats_hbm.at[pl.ds(0, M_CHUNK * FAN0)],
                              buf[b], msem[b]).wait()

    fire_m(0, 0)
    fire_m(1, 1)

    def reduce_chunk(b):
        for r in range(M_CHUNK):
            base = r * FAN0
            accs = [buf[b][base, pl.ds(16 * j, 16)] for j in range(D // 16)]

            def acc_body(k, a):
                return tuple(a[j] + buf[b][base + k, pl.ds(16 * j, 16)]
                             for j in range(D // 16))

            accs = lax.fori_loop(1, FAN0, acc_body, tuple(accs))
            for j in range(D // 16):
                obuf[r, pl.ds(16 * j, 16)] = accs[j] * (1.0 / FAN0)

    def write_chunk(c):
        row = m_row0 + c * M_CHUNK

        @pl.when(row < B)
        def _():
            pltpu.sync_copy(obuf, m1_out.at[pl.ds(row, M_CHUNK)])

        @pl.when(row >= B)
        def _():
            pltpu.sync_copy(obuf, m2_out.at[pl.ds(row - B, M_CHUNK)])

    def pair_body(g, carry):
        for b in range(2):
            c = 2 * g + b
            drain_m(b)
            reduce_chunk(b)
            write_chunk(c)

            @pl.when(c + 2 < M_CHUNKS)
            def _():
                fire_m(c + 2, b)
        return carry

    lax.fori_loop(0, M_CHUNKS // 2, pair_body, 0)

    pltpu.make_async_copy(feats_hbm.at[pl.ds(0, HR_PER_W)], bufh, hsem).wait()
    pltpu.sync_copy(bufh, hp2_out.at[pl.ds(h_row0, HR_PER_W)])


@jax.jit
def _sc_gather(feats, n1, n2n, idx_h):
    mesh = plsc.VectorSubcoreMesh(core_axis_name="c", subcore_axis_name="s",
                                  num_cores=NC, num_subcores=NS)
    return pl.kernel(
        _sc_gather_body,
        out_type=(
            jax.ShapeDtypeStruct((B, D), jnp.float32),
            jax.ShapeDtypeStruct((B * FAN1, D), jnp.float32),
            jax.ShapeDtypeStruct((H_ROWS, D), jnp.float32),
        ),
        mesh=mesh,
        scratch_types=[
            pltpu.VMEM((M_CHUNK, FAN0), jnp.int32),
            pltpu.VMEM((M_CHUNK, FAN0), jnp.int32),
            pltpu.VMEM((M_CHUNK * FAN0, D), jnp.float32),
            pltpu.VMEM((M_CHUNK * FAN0, D), jnp.float32),
            pltpu.VMEM((M_CHUNK, D), jnp.float32),
            pltpu.VMEM((HR_PER_W,), jnp.int32),
            pltpu.VMEM((HR_PER_W, D), jnp.float32),
            pltpu.SemaphoreType.DMA,
            pltpu.SemaphoreType.DMA,
            pltpu.SemaphoreType.DMA,
        ],
    )(feats, n1, n2n, idx_h)


def _tc_dense_body(x_ref, m1_ref, hp2_ref, m2_ref, w0_ref, b0_ref,
                   w1_ref, b1_ref, out_ref):
    def lin(a, b, w_ref, bias):
        return jnp.maximum(
            jnp.dot(a, w_ref[:D], preferred_element_type=jnp.float32)
            + jnp.dot(b, w_ref[D:], preferred_element_type=jnp.float32)
            + bias, 0.0)

    h0 = lin(x_ref[...], m1_ref[...], w0_ref, b0_ref[...])
    h2 = lin(hp2_ref[...], m2_ref[...], w0_ref, b0_ref[...])
    na = jnp.mean(h2.reshape(B, FAN1, HID), axis=1)
    out_ref[...] = lin(h0, na, w1_ref, b1_ref[...])


@jax.jit
def _tc_dense(x, m1, hp2, m2, W0, b0, W1, b1):
    return pl.pallas_call(
        _tc_dense_body,
        out_shape=jax.ShapeDtypeStruct((B, HID), jnp.float32),
    )(x, m1, hp2, m2, W0, b0.reshape(1, HID), W1, b1.reshape(1, HID))


def kernel(x, nodes, feats, neighs1, neighs2, neighs2_neighs, W0, b0, W1, b1):
    idx_h = neighs2.reshape(-1).astype(jnp.int32)
    m1, m2, hp2 = _sc_gather(feats, neighs1.astype(jnp.int32),
                             neighs2_neighs.astype(jnp.int32), idx_h)
    out = _tc_dense(x, m1, hp2, m2, W0, b0, W1, b1)
    return out.reshape(B, 1, HID)


# speedup vs baseline: 1.1417x; 1.1300x over previous
import jax
import jax.numpy as jnp
from jax import lax
from jax.experimental import pallas as pl
from jax.experimental.pallas import tpu as pltpu
from jax.experimental.pallas import tpu_sc as plsc

B = 1024
FAN0 = 25
FAN1 = 10
D = 128
HID = 128

NC = 2
NS = 16
NW = NC * NS

M_ROWS = B + B * FAN1
MR_PER_W = M_ROWS // NW
M_CHUNK = 8
M_CHUNKS = MR_PER_W // M_CHUNK
N1_CHUNKS = B // M_CHUNK

H_ROWS = B * FAN1
HR_PER_W = H_ROWS // NW
H_GLEN = 64
H_GATHERS = HR_PER_W // H_GLEN


def _sc_gather_body(feats_hbm, n1_hbm, n2n_hbm, idxh_hbm,
                    m1_out, m2_out, hp2_out,
                    idxv0, idxv1, idxv2, idxv3, buf0, buf1, obuf0, obuf1,
                    idxh, bufh,
                    isem0, isem1, isem2, isem3, msem0, msem1,
                    wsem0, wsem1, hsem):
    wid = lax.axis_index("s") * NC + lax.axis_index("c")
    idxv = (idxv0, idxv1, idxv2, idxv3)
    isem = (isem0, isem1, isem2, isem3)
    buf = (buf0, buf1)
    msem = (msem0, msem1)
    obuf = (obuf0, obuf1)
    wsem = (wsem0, wsem1)

    m_row0 = wid * MR_PER_W
    h_row0 = wid * HR_PER_W

    pltpu.sync_copy(idxh_hbm.at[pl.ds(h_row0, HR_PER_W)], idxh)
    for j in range(H_GATHERS):
        pltpu.async_copy(feats_hbm.at[idxh.at[pl.ds(j * H_GLEN, H_GLEN)]],
                         bufh.at[pl.ds(j * H_GLEN, H_GLEN)], hsem)

    def start_idx(c, q):
        g = wid * M_CHUNKS + c

        @pl.when(g < N1_CHUNKS)
        def _():
            pltpu.async_copy(n1_hbm.at[pl.ds(g * M_CHUNK, M_CHUNK)],
                             idxv[q], isem[q])

        @pl.when(g >= N1_CHUNKS)
        def _():
            pltpu.async_copy(n2n_hbm.at[pl.ds(g * M_CHUNK - B, M_CHUNK)],
                             idxv[q], isem[q])

    def fire_m(c, q, b):
        pltpu.make_async_copy(n1_hbm.at[pl.ds(0, M_CHUNK)], idxv[q],
                              isem[q]).wait()
        for r in range(M_CHUNK):
            pltpu.async_copy(feats_hbm.at[idxv[q].at[r]],
                             buf[b].at[pl.ds(r * FAN0, FAN0)], msem[b])

    def drain_m(b):
        pltpu.make_async_copy(feats_hbm.at[pl.ds(0, M_CHUNK * FAN0)],
                              buf[b], msem[b]).wait()

    def reduce_chunk(b):
        for r in range(M_CHUNK):
            base = r * FAN0
            accs = [buf[b][base, pl.ds(16 * j, 16)] for j in range(D // 16)]

            def acc_body(k, a):
                return tuple(a[j] + buf[b][base + k, pl.ds(16 * j, 16)]
                             for j in range(D // 16))

            accs = lax.fori_loop(1, FAN0, acc_body, tuple(accs))
            for j in range(D // 16):
                obuf[b][r, pl.ds(16 * j, 16)] = accs[j] * (1.0 / FAN0)

    def write_chunk(c, b):
        row = m_row0 + c * M_CHUNK

        @pl.when(row < B)
        def _():
            pltpu.async_copy(obuf[b], m1_out.at[pl.ds(row, M_CHUNK)], wsem[b])

        @pl.when(row >= B)
        def _():
            pltpu.async_copy(obuf[b], m2_out.at[pl.ds(row - B, M_CHUNK)],
                             wsem[b])

    for c in range(4):
        start_idx(c, c)
    fire_m(0, 0, 0)
    fire_m(1, 1, 1)

    def quad_body(g, carry):
        for u in range(4):
            c = 4 * g + u
            b = u % 2
            drain_m(b)

            @pl.when(c >= 2)
            def _():
                pltpu.make_async_copy(obuf[b], m2_out.at[pl.ds(0, M_CHUNK)],
                                      wsem[b]).wait()

            reduce_chunk(b)
            write_chunk(c, b)

            @pl.when(c + 2 < M_CHUNKS)
            def _():
                fire_m(c + 2, (u + 2) % 4, b)

            @pl.when(c + 4 < M_CHUNKS)
            def _():
                start_idx(c + 4, u)
        return carry

    lax.fori_loop(0, M_CHUNKS // 4, quad_body, 0)

    for b in range(2):
        pltpu.make_async_copy(obuf[b], m2_out.at[pl.ds(0, M_CHUNK)],
                              wsem[b]).wait()

    pltpu.make_async_copy(feats_hbm.at[pl.ds(0, HR_PER_W)], bufh, hsem).wait()
    pltpu.sync_copy(bufh, hp2_out.at[pl.ds(h_row0, HR_PER_W)])


@jax.jit
def _sc_gather(feats, n1, n2n, idx_h):
    mesh = plsc.VectorSubcoreMesh(core_axis_name="c", subcore_axis_name="s",
                                  num_cores=NC, num_subcores=NS)
    return pl.kernel(
        _sc_gather_body,
        out_type=(
            jax.ShapeDtypeStruct((B, D), jnp.float32),
            jax.ShapeDtypeStruct((B * FAN1, D), jnp.float32),
            jax.ShapeDtypeStruct((H_ROWS, D), jnp.float32),
        ),
        mesh=mesh,
        scratch_types=[
            pltpu.VMEM((M_CHUNK, FAN0), jnp.int32),
            pltpu.VMEM((M_CHUNK, FAN0), jnp.int32),
            pltpu.VMEM((M_CHUNK, FAN0), jnp.int32),
            pltpu.VMEM((M_CHUNK, FAN0), jnp.int32),
            pltpu.VMEM((M_CHUNK * FAN0, D), jnp.float32),
            pltpu.VMEM((M_CHUNK * FAN0, D), jnp.float32),
            pltpu.VMEM((M_CHUNK, D), jnp.float32),
            pltpu.VMEM((M_CHUNK, D), jnp.float32),
            pltpu.VMEM((HR_PER_W,), jnp.int32),
            pltpu.VMEM((HR_PER_W, D), jnp.float32),
            pltpu.SemaphoreType.DMA,
            pltpu.SemaphoreType.DMA,
            pltpu.SemaphoreType.DMA,
            pltpu.SemaphoreType.DMA,
            pltpu.SemaphoreType.DMA,
            pltpu.SemaphoreType.DMA,
            pltpu.SemaphoreType.DMA,
            pltpu.SemaphoreType.DMA,
            pltpu.SemaphoreType.DMA,
        ],
    )(feats, n1, n2n, idx_h)


def _tc_dense_body(x_ref, m1_ref, hp2_ref, m2_ref, w0_ref, b0_ref,
                   w1_ref, b1_ref, out_ref):
    def lin(a, b, w_ref, bias):
        return jnp.maximum(
            jnp.dot(a, w_ref[:D], preferred_element_type=jnp.float32)
            + jnp.dot(b, w_ref[D:], preferred_element_type=jnp.float32)
            + bias, 0.0)

    h0 = lin(x_ref[...], m1_ref[...], w0_ref, b0_ref[...])
    h2 = lin(hp2_ref[...], m2_ref[...], w0_ref, b0_ref[...])
    na = jnp.mean(h2.reshape(B, FAN1, HID), axis=1)
    out_ref[...] = lin(h0, na, w1_ref, b1_ref[...])


@jax.jit
def _tc_dense(x, m1, hp2, m2, W0, b0, W1, b1):
    return pl.pallas_call(
        _tc_dense_body,
        out_shape=jax.ShapeDtypeStruct((B, HID), jnp.float32),
    )(x, m1, hp2, m2, W0, b0.reshape(1, HID), W1, b1.reshape(1, HID))


def kernel(x, nodes, feats, neighs1, neighs2, neighs2_neighs, W0, b0, W1, b1):
    idx_h = neighs2.reshape(-1).astype(jnp.int32)
    m1, m2, hp2 = _sc_gather(feats, neighs1.astype(jnp.int32),
                             neighs2_neighs.astype(jnp.int32), idx_h)
    out = _tc_dense(x, m1, hp2, m2, W0, b0, W1, b1)
    return out.reshape(B, 1, HID)


# speedup vs baseline: 1.1521x; 1.0091x over previous
import jax
import jax.numpy as jnp
from jax import lax
from jax.experimental import pallas as pl
from jax.experimental.pallas import tpu as pltpu
from jax.experimental.pallas import tpu_sc as plsc

B = 1024
FAN0 = 25
FAN1 = 10
D = 128
HID = 128

NC = 2
NS = 16
NW = NC * NS

M_ROWS = B + B * FAN1
MR_PER_W = M_ROWS // NW
M_CHUNK = 8
M_CHUNKS = MR_PER_W // M_CHUNK
N1_CHUNKS = B // M_CHUNK

H_ROWS = B * FAN1
HR_PER_W = H_ROWS // NW
H_GLEN = 64
H_GATHERS = HR_PER_W // H_GLEN


def _sc_gather_body(feats_hbm, n1_hbm, n2n_hbm, idxh_hbm,
                    m1_out, m2_out, hp2_out,
                    idxv0, idxv1, idxv2, idxv3, buf0, buf1, obuf0, obuf1,
                    idxh, bufh,
                    isem0, isem1, isem2, isem3, msem0, msem1,
                    wsem0, wsem1, hsem):
    wid = lax.axis_index("s") * NC + lax.axis_index("c")
    idxv = (idxv0, idxv1, idxv2, idxv3)
    isem = (isem0, isem1, isem2, isem3)
    buf = (buf0, buf1)
    msem = (msem0, msem1)
    obuf = (obuf0, obuf1)
    wsem = (wsem0, wsem1)

    m_row0 = wid * MR_PER_W
    h_row0 = wid * HR_PER_W

    pltpu.sync_copy(idxh_hbm.at[pl.ds(h_row0, HR_PER_W)], idxh)
    for j in range(H_GATHERS):
        pltpu.async_copy(feats_hbm.at[idxh.at[pl.ds(j * H_GLEN, H_GLEN)]],
                         bufh.at[pl.ds(j * H_GLEN, H_GLEN)], hsem)

    def start_idx(c, q):
        g = wid * M_CHUNKS + c

        @pl.when(g < N1_CHUNKS)
        def _():
            pltpu.async_copy(n1_hbm.at[pl.ds(g * M_CHUNK, M_CHUNK)],
                             idxv[q], isem[q])

        @pl.when(g >= N1_CHUNKS)
        def _():
            pltpu.async_copy(n2n_hbm.at[pl.ds(g * M_CHUNK - B, M_CHUNK)],
                             idxv[q], isem[q])

    def fire_m(c, q, b):
        pltpu.make_async_copy(n1_hbm.at[pl.ds(0, M_CHUNK)], idxv[q],
                              isem[q]).wait()
        for r in range(M_CHUNK):
            pltpu.async_copy(feats_hbm.at[idxv[q].at[r]],
                             buf[b].at[pl.ds(r * FAN0, FAN0)], msem[b])

    def drain_m(b):
        pltpu.make_async_copy(feats_hbm.at[pl.ds(0, M_CHUNK * FAN0)],
                              buf[b], msem[b]).wait()

    def reduce_chunk(b):
        for r in range(M_CHUNK):
            base = r * FAN0
            accs = [buf[b][base, pl.ds(16 * j, 16)] for j in range(D // 16)]

            def acc_body(k, a):
                return tuple(a[j] + buf[b][base + k, pl.ds(16 * j, 16)]
                             for j in range(D // 16))

            accs = lax.fori_loop(1, FAN0, acc_body, tuple(accs))
            for j in range(D // 16):
                obuf[b][r, pl.ds(16 * j, 16)] = accs[j] * (1.0 / FAN0)

    def write_chunk(c, b):
        row = m_row0 + c * M_CHUNK

        @pl.when(row < B)
        def _():
            pltpu.async_copy(obuf[b], m1_out.at[pl.ds(row, M_CHUNK)], wsem[b])

        @pl.when(row >= B)
        def _():
            pltpu.async_copy(obuf[b], m2_out.at[pl.ds(row - B, M_CHUNK)],
                             wsem[b])

    for c in range(4):
        start_idx(c, c)
    fire_m(0, 0, 0)
    fire_m(1, 1, 1)

    pltpu.make_async_copy(feats_hbm.at[pl.ds(0, HR_PER_W)], bufh, hsem).wait()
    pltpu.async_copy(bufh, hp2_out.at[pl.ds(h_row0, HR_PER_W)], hsem)

    def quad_body(g, carry):
        for u in range(4):
            c = 4 * g + u
            b = u % 2
            drain_m(b)

            @pl.when(c >= 2)
            def _():
                pltpu.make_async_copy(obuf[b], m2_out.at[pl.ds(0, M_CHUNK)],
                                      wsem[b]).wait()

            reduce_chunk(b)
            write_chunk(c, b)

            @pl.when(c + 2 < M_CHUNKS)
            def _():
                fire_m(c + 2, (u + 2) % 4, b)

            @pl.when(c + 4 < M_CHUNKS)
            def _():
                start_idx(c + 4, u)
        return carry

    lax.fori_loop(0, M_CHUNKS // 4, quad_body, 0)

    for b in range(2):
        pltpu.make_async_copy(obuf[b], m2_out.at[pl.ds(0, M_CHUNK)],
                              wsem[b]).wait()

    pltpu.make_async_copy(bufh, hp2_out.at[pl.ds(h_row0, HR_PER_W)],
                          hsem).wait()


@jax.jit
def _sc_gather(feats, n1, n2n, idx_h):
    mesh = plsc.VectorSubcoreMesh(core_axis_name="c", subcore_axis_name="s",
                                  num_cores=NC, num_subcores=NS)
    return pl.kernel(
        _sc_gather_body,
        out_type=(
            jax.ShapeDtypeStruct((B, D), jnp.float32),
            jax.ShapeDtypeStruct((B * FAN1, D), jnp.float32),
            jax.ShapeDtypeStruct((H_ROWS, D), jnp.float32),
        ),
        mesh=mesh,
        scratch_types=[
            pltpu.VMEM((M_CHUNK, FAN0), jnp.int32),
            pltpu.VMEM((M_CHUNK, FAN0), jnp.int32),
            pltpu.VMEM((M_CHUNK, FAN0), jnp.int32),
            pltpu.VMEM((M_CHUNK, FAN0), jnp.int32),
            pltpu.VMEM((M_CHUNK * FAN0, D), jnp.float32),
            pltpu.VMEM((M_CHUNK * FAN0, D), jnp.float32),
            pltpu.VMEM((M_CHUNK, D), jnp.float32),
            pltpu.VMEM((M_CHUNK, D), jnp.float32),
            pltpu.VMEM((HR_PER_W,), jnp.int32),
            pltpu.VMEM((HR_PER_W, D), jnp.float32),
            pltpu.SemaphoreType.DMA,
            pltpu.SemaphoreType.DMA,
            pltpu.SemaphoreType.DMA,
            pltpu.SemaphoreType.DMA,
            pltpu.SemaphoreType.DMA,
            pltpu.SemaphoreType.DMA,
            pltpu.SemaphoreType.DMA,
            pltpu.SemaphoreType.DMA,
            pltpu.SemaphoreType.DMA,
        ],
    )(feats, n1, n2n, idx_h)


def _tc_dense_body(x_ref, m1_ref, hp2_ref, m2_ref, w0_ref, b0_ref,
                   w1_ref, b1_ref, out_ref):
    def lin(a, b, w_ref, bias):
        return jnp.maximum(
            jnp.dot(a, w_ref[:D], preferred_element_type=jnp.float32)
            + jnp.dot(b, w_ref[D:], preferred_element_type=jnp.float32)
            + bias, 0.0)

    h0 = lin(x_ref[...], m1_ref[...], w0_ref, b0_ref[...])
    h2 = lin(hp2_ref[...], m2_ref[...], w0_ref, b0_ref[...])
    na = jnp.mean(h2.reshape(B, FAN1, HID), axis=1)
    out_ref[...] = lin(h0, na, w1_ref, b1_ref[...])


@jax.jit
def _tc_dense(x, m1, hp2, m2, W0, b0, W1, b1):
    return pl.pallas_call(
        _tc_dense_body,
        out_shape=jax.ShapeDtypeStruct((B, HID), jnp.float32),
    )(x, m1, hp2, m2, W0, b0.reshape(1, HID), W1, b1.reshape(1, HID))


def kernel(x, nodes, feats, neighs1, neighs2, neighs2_neighs, W0, b0, W1, b1):
    idx_h = neighs2.reshape(-1).astype(jnp.int32)
    m1, m2, hp2 = _sc_gather(feats, neighs1.astype(jnp.int32),
                             neighs2_neighs.astype(jnp.int32), idx_h)
    out = _tc_dense(x, m1, hp2, m2, W0, b0, W1, b1)
    return out.reshape(B, 1, HID)
